# Initial kernel scaffold; baseline (speedup 1.0000x reference)
import jax
import jax.numpy as jnp
from jax.experimental import pallas as pl


def kernel(discrete_actions, table):
    raise NotImplementedError("write your pallas kernel here")


# speedup vs baseline: 13.7306x; 13.7306x over previous
import functools

import jax
import jax.numpy as jnp
from jax import lax
from jax.experimental import pallas as pl
from jax.experimental.pallas import tpu as pltpu
from jax.experimental.pallas import tpu_sc as plsc

_NUM_AXES = 14
_MAX_BINS = 1024


def kernel(discrete_actions, table):
    B, T, A = discrete_actions.shape
    _, _, D = table.shape
    N = B * T * A

    idx_flat = discrete_actions.reshape(N).astype(jnp.int32)
    table_flat = table.reshape(A * _MAX_BINS, D)

    info = plsc.get_sparse_core_info()
    NC, NS, L = info.num_cores, info.num_subcores, info.num_lanes
    NW = NC * NS
    per_w = N // NW
    CHUNK = 128
    n_chunks = per_w // CHUNK

    mesh = plsc.VectorSubcoreMesh(core_axis_name="c", subcore_axis_name="s")

    @functools.partial(
        pl.kernel,
        out_type=jax.ShapeDtypeStruct((N, D), jnp.float32),
        mesh=mesh,
        compiler_params=pltpu.CompilerParams(use_tc_tiling_on_sc=False),
        scratch_types=[
            pltpu.VMEM((CHUNK,), jnp.int32),
            pltpu.VMEM((CHUNK, D), jnp.float32),
            pltpu.SemaphoreType.DMA,
        ],
    )
    def _gather(idx_hbm, tab_hbm, out_hbm, fidx_v, rows_v, sem):
        wid = lax.axis_index("s") * NC + lax.axis_index("c")
        wbase = wid * per_w

        def body(c, _):
            base = wbase + c * CHUNK
            pltpu.sync_copy(idx_hbm.at[pl.ds(base, CHUNK)], fidx_v)
            for j in range(CHUNK // L):
                lanes = lax.iota(jnp.int32, L) + (base + j * L)
                axis = lax.rem(lanes, A)
                sl = pl.ds(j * L, L)
                fidx_v[sl] = fidx_v[sl] + axis * _MAX_BINS
            pltpu.async_copy(tab_hbm.at[fidx_v], rows_v, sem).wait()
            pltpu.sync_copy(rows_v, out_hbm.at[pl.ds(base, CHUNK)])
            return 0

        lax.fori_loop(0, n_chunks, body, 0)

    out = _gather(idx_flat, table_flat)
    return out.reshape(B, T, A, D)


# speedup vs baseline: 18.2214x; 1.3271x over previous
import functools

import jax
import jax.numpy as jnp
from jax import lax
from jax.experimental import pallas as pl
from jax.experimental.pallas import tpu as pltpu
from jax.experimental.pallas import tpu_sc as plsc

_NUM_AXES = 14
_MAX_BINS = 1024


def kernel(discrete_actions, table):
    B, T, A = discrete_actions.shape
    _, _, D = table.shape
    N = B * T * A

    idx2d = discrete_actions.reshape(N // 128, 128).astype(jnp.int32)
    table_flat = table.reshape(A * _MAX_BINS, D)

    info = plsc.get_sparse_core_info()
    NC, NS, L = info.num_cores, info.num_subcores, info.num_lanes
    NW = NC * NS
    per_w = N // NW
    SUB = 128
    NSUB = 7
    SUPER = SUB * NSUB
    n_super = per_w // SUPER
    n_pairs = n_super // 2

    offs = ((jnp.arange(SUPER, dtype=jnp.int32) % A) * _MAX_BINS).reshape(NSUB, SUB)

    mesh = plsc.VectorSubcoreMesh(core_axis_name="c", subcore_axis_name="s")

    @functools.partial(
        pl.kernel,
        out_type=jax.ShapeDtypeStruct((N, D), jnp.float32),
        mesh=mesh,
        compiler_params=pltpu.CompilerParams(use_tc_tiling_on_sc=False),
        scratch_types=[
            pltpu.VMEM((2, NSUB, SUB), jnp.int32),
            pltpu.VMEM((NSUB, SUB), jnp.int32),
            pltpu.VMEM((2, SUPER, D), jnp.float32),
            pltpu.SemaphoreType.DMA,
            pltpu.SemaphoreType.DMA,
        ],
    )
    def _gather(idx_hbm, tab_hbm, offs_hbm, out_hbm, fidx_v, offs_v, rows_v,
                gsem, osem):
        wid = lax.axis_index("s") * NC + lax.axis_index("c")
        wbase = wid * per_w
        pltpu.sync_copy(offs_hbm, offs_v)

        def half(base, p, first):
            pltpu.sync_copy(idx_hbm.at[pl.ds(base // SUB, NSUB)], fidx_v.at[p])
            for j in range(NSUB):
                for v in range(SUB // L):
                    sl = pl.ds(v * L, L)
                    fidx_v[p, j, sl] = fidx_v[p, j, sl] + offs_v[j, sl]
            for j in range(NSUB):
                pltpu.async_copy(
                    tab_hbm.at[fidx_v.at[p, j]],
                    rows_v.at[p, pl.ds(j * SUB, SUB)],
                    gsem,
                )
            for j in range(NSUB):
                pltpu.make_async_copy(
                    tab_hbm.at[fidx_v.at[p, j]],
                    rows_v.at[p, pl.ds(j * SUB, SUB)],
                    gsem,
                ).wait()
            @pl.when(jnp.logical_not(first))
            def _():
                pltpu.make_async_copy(
                    rows_v.at[1 - p],
                    out_hbm.at[pl.ds(base - SUPER, SUPER)],
                    osem,
                ).wait()
            pltpu.async_copy(
                rows_v.at[p],
                out_hbm.at[pl.ds(base, SUPER)],
                osem,
            )

        def body(i, _):
            base = wbase + i * (2 * SUPER)
            half(base, 0, i == 0)
            half(base + SUPER, 1, False)
            return 0

        lax.fori_loop(0, n_pairs, body, 0)
        pltpu.make_async_copy(
            rows_v.at[1],
            out_hbm.at[pl.ds(wbase + per_w - SUPER, SUPER)],
            osem,
        ).wait()

    out = _gather(idx2d, table_flat, offs)
    return out.reshape(B, T, A, D)


# speedup vs baseline: 19.4183x; 1.0657x over previous
import functools

import jax
import jax.numpy as jnp
from jax import lax
from jax.experimental import pallas as pl
from jax.experimental.pallas import tpu as pltpu
from jax.experimental.pallas import tpu_sc as plsc

_NUM_AXES = 14
_MAX_BINS = 1024


def kernel(discrete_actions, table):
    B, T, A = discrete_actions.shape
    _, _, D = table.shape
    N = B * T * A

    idx2d = discrete_actions.reshape(N // 112, 112).astype(jnp.int32)
    table_flat = table.reshape(A * _MAX_BINS, D)

    info = plsc.get_sparse_core_info()
    NC, NS, L = info.num_cores, info.num_subcores, info.num_lanes
    NW = NC * NS
    per_w = N // NW
    SUB = 112
    NSUB = 4
    SUPER = SUB * NSUB
    n_super = per_w // SUPER
    n_pairs = n_super // 2

    offs = ((jnp.arange(SUPER, dtype=jnp.int32) % A) * _MAX_BINS).reshape(NSUB, SUB)

    mesh = plsc.VectorSubcoreMesh(core_axis_name="c", subcore_axis_name="s")

    @functools.partial(
        pl.kernel,
        out_type=jax.ShapeDtypeStruct((N, D), jnp.float32),
        mesh=mesh,
        compiler_params=pltpu.CompilerParams(use_tc_tiling_on_sc=False),
        scratch_types=[
            pltpu.VMEM((2, NSUB, SUB), jnp.int32),
            pltpu.VMEM((NSUB, SUB), jnp.int32),
            pltpu.VMEM((2, SUPER, D), jnp.float32),
            pltpu.VMEM_SHARED((A * _MAX_BINS, D), jnp.float32),
            pltpu.SemaphoreType.DMA,
            pltpu.SemaphoreType.DMA,
        ],
    )
    def _gather(idx_hbm, tab_hbm, offs_hbm, out_hbm, fidx_v, offs_v, rows_v,
                tab_s, gsem, osem):
        wid = lax.axis_index("s") * NC + lax.axis_index("c")
        wbase = wid * per_w
        sid = lax.axis_index("s")
        nrows = A * _MAX_BINS // NS
        pltpu.sync_copy(tab_hbm.at[pl.ds(sid * nrows, nrows)],
                        tab_s.at[pl.ds(sid * nrows, nrows)])
        plsc.subcore_barrier()
        pltpu.sync_copy(offs_hbm, offs_v)

        def half(base, p, first):
            pltpu.sync_copy(idx_hbm.at[pl.ds(base // SUB, NSUB)], fidx_v.at[p])
            for j in range(NSUB):
                for v in range(SUB // L):
                    sl = pl.ds(v * L, L)
                    fidx_v[p, j, sl] = fidx_v[p, j, sl] + offs_v[j, sl]
            for j in range(NSUB):
                pltpu.async_copy(
                    tab_s.at[fidx_v.at[p, j]],
                    rows_v.at[p, pl.ds(j * SUB, SUB)],
                    gsem,
                )
            for j in range(NSUB):
                pltpu.make_async_copy(
                    tab_s.at[fidx_v.at[p, j]],
                    rows_v.at[p, pl.ds(j * SUB, SUB)],
                    gsem,
                ).wait()
            @pl.when(jnp.logical_not(first))
            def _():
                pltpu.make_async_copy(
                    rows_v.at[1 - p],
                    out_hbm.at[pl.ds(base - SUPER, SUPER)],
                    osem,
                ).wait()
            pltpu.async_copy(
                rows_v.at[p],
                out_hbm.at[pl.ds(base, SUPER)],
                osem,
            )

        def body(i, _):
            base = wbase + i * (2 * SUPER)
            half(base, 0, i == 0)
            half(base + SUPER, 1, False)
            return 0

        lax.fori_loop(0, n_pairs, body, 0)
        pltpu.make_async_copy(
            rows_v.at[1],
            out_hbm.at[pl.ds(wbase + per_w - SUPER, SUPER)],
            osem,
        ).wait()

    out = _gather(idx2d, table_flat, offs)
    return out.reshape(B, T, A, D)


# speedup vs baseline: 24.0176x; 1.2369x over previous
import functools

import jax
import jax.numpy as jnp
from jax import lax
from jax.experimental import pallas as pl
from jax.experimental.pallas import tpu as pltpu
from jax.experimental.pallas import tpu_sc as plsc


def kernel(discrete_actions, table):
    B, T, A = discrete_actions.shape
    _, MB, D = table.shape

    info = plsc.get_sparse_core_info()
    NC, NS, L = info.num_cores, info.num_subcores, info.num_lanes
    NW = NC * NS
    BC = B // NW

    idxT = discrete_actions.transpose(2, 0, 1)
    idxT = idxT.reshape(A, NW, BC, T).transpose(0, 1, 3, 2)
    idx_lin = idxT.reshape(A * NW * T * BC).astype(jnp.int32)

    tab_lin = jnp.swapaxes(table, 1, 2).reshape(A * D * MB)

    mesh = plsc.VectorSubcoreMesh(core_axis_name="c", subcore_axis_name="s")

    @functools.partial(
        pl.kernel,
        out_type=jax.ShapeDtypeStruct((T, A, D, B), jnp.float32),
        mesh=mesh,
        compiler_params=pltpu.CompilerParams(needs_layout_passes=False),
        scratch_types=[
            pltpu.VMEM((D * MB,), jnp.float32),
            pltpu.VMEM((T * BC,), jnp.int32),
            pltpu.VMEM((2, D, BC), jnp.float32),
            pltpu.SemaphoreType.DMA,
        ],
    )
    def _gather(idx_hbm, tab_hbm, out_hbm, tab_v, idxc_v, outbuf_v, osem):
        wid = lax.axis_index("s") * NC + lax.axis_index("c")

        def wait_one_block():
            pltpu.make_async_copy(
                outbuf_v.at[0],
                out_hbm.at[0, 0, :, pl.ds(wid * BC, BC)],
                osem,
            ).wait()

        def half(a, t, p):
            g = a * T + t
            @pl.when(g >= 2)
            def _():
                wait_one_block()
            bins = [idxc_v[pl.ds(t * BC + c * L, L)] for c in range(BC // L)]
            for d in range(D):
                refd = tab_v.at[pl.ds(d * MB, MB)]
                for c in range(BC // L):
                    outbuf_v[p, d, pl.ds(c * L, L)] = plsc.load_gather(
                        refd, [bins[c]])
            pltpu.async_copy(
                outbuf_v.at[p],
                out_hbm.at[t, a, :, pl.ds(wid * BC, BC)],
                osem,
            )

        def a_body(a, _):
            pltpu.sync_copy(tab_hbm.at[pl.ds(a * D * MB, D * MB)], tab_v)
            pltpu.sync_copy(
                idx_hbm.at[pl.ds((a * NW + wid) * T * BC, T * BC)], idxc_v)

            def t_body(tp, _):
                half(a, 2 * tp, 0)
                half(a, 2 * tp + 1, 1)
                return 0

            lax.fori_loop(0, T // 2, t_body, 0)
            return 0

        lax.fori_loop(0, A, a_body, 0)
        wait_one_block()
        wait_one_block()

    out_t = _gather(idx_lin, tab_lin)
    return jnp.transpose(out_t, (3, 0, 1, 2))


# speedup vs baseline: 62.8619x; 2.6173x over previous
import functools

import jax
import jax.numpy as jnp
from jax import lax
from jax.experimental import pallas as pl
from jax.experimental.pallas import tpu as pltpu
from jax.experimental.pallas import tpu_sc as plsc


def kernel(discrete_actions, table):
    B, T, A = discrete_actions.shape
    _, MB, D = table.shape

    info = plsc.get_sparse_core_info()
    NC, NS, L = info.num_cores, info.num_subcores, info.num_lanes
    NW = NC * NS
    BC = B // NW

    idxT = discrete_actions.transpose(2, 0, 1)
    idxT = idxT.reshape(A, NW, BC, T).transpose(0, 1, 3, 2)
    idx_lin = idxT.reshape(A * NW * T * BC).astype(jnp.int32)

    tab_lin = jnp.swapaxes(table, 1, 2).reshape(A * D * MB)

    mesh = plsc.VectorSubcoreMesh(core_axis_name="c", subcore_axis_name="s")

    @functools.partial(
        pl.kernel,
        out_type=jax.ShapeDtypeStruct((T, A, D, B), jnp.float32),
        mesh=mesh,
        compiler_params=pltpu.CompilerParams(needs_layout_passes=False),
        scratch_types=[
            pltpu.VMEM((D * MB,), jnp.float32),
            pltpu.VMEM((T * BC,), jnp.int32),
            pltpu.VMEM((4, D, BC), jnp.float32),
            pltpu.SemaphoreType.DMA,
        ],
    )
    def _gather(idx_hbm, tab_hbm, out_hbm, tab_v, idxc_v, outbuf_v, osem):
        wid = lax.axis_index("s") * NC + lax.axis_index("c")

        def wait_one_block():
            pltpu.make_async_copy(
                outbuf_v.at[0],
                out_hbm.at[0, 0, :, pl.ds(wid * BC, BC)],
                osem,
            ).wait()

        def block(g, p):
            a = g // T
            t = g - a * T
            @pl.when(t == 0)
            def _():
                pltpu.sync_copy(tab_hbm.at[pl.ds(a * D * MB, D * MB)], tab_v)
                pltpu.sync_copy(
                    idx_hbm.at[pl.ds((a * NW + wid) * T * BC, T * BC)], idxc_v)
            @pl.when(g >= 4)
            def _():
                wait_one_block()
            NB = BC // L
            bins = [idxc_v[pl.ds(t * BC + c * L, L)] for c in range(NB)]

            def load_row(d):
                ref = tab_v.at[pl.ds(d * MB, MB)]
                return [plsc.load_gather(ref, [b]) for b in bins]

            prev = load_row(0)
            for d in range(1, D):
                ref = tab_v.at[pl.ds(d * MB, MB)]
                cur = []
                for c in range(NB):
                    cur.append(plsc.load_gather(ref, [bins[c]]))
                    outbuf_v[p, d - 1, pl.ds(c * L, L)] = prev[c]
                prev = cur
            for c in range(NB):
                outbuf_v[p, D - 1, pl.ds(c * L, L)] = prev[c]
            pltpu.async_copy(
                outbuf_v.at[p],
                out_hbm.at[t, a, :, pl.ds(wid * BC, BC)],
                osem,
            )

        def g_body(q, _):
            for i in range(4):
                block(4 * q + i, i)
            return 0

        lax.fori_loop(0, A * T // 4, g_body, 0)
        for _ in range(4):
            wait_one_block()

    out_t = _gather(idx_lin, tab_lin)
    return jnp.transpose(out_t, (3, 0, 1, 2))
